# baseline (speedup 1.0000x reference)
import jax
import jax.numpy as jnp
from jax import lax
from jax.experimental import pallas as pl
from jax.experimental.pallas import tpu as pltpu
from jax.experimental.pallas import tpu_sc as plsc

NC = 2
NS = 16
NW = NC * NS
K = 128
F32 = jnp.float32


def _mesh():
    return plsc.VectorSubcoreMesh(core_axis_name="c", subcore_axis_name="s",
                                  num_cores=NC, num_subcores=NS)


def _cdiv(a, b):
    return -(-a // b)


def _round_up(a, b):
    return _cdiv(a, b) * b


def _sc_degree(ei, n_pad, d):
    c_chunks = ei.shape[2]
    rps = n_pad // NS

    def body(ei_hbm, out_hbm, zeros_v, ones_s, ones_d, idxa, idxb, acc_sh,
             semi):
        cid = lax.axis_index("c")
        sid = lax.axis_index("s")

        @pl.loop(0, 32)
        def _(i):
            @pl.loop(0, d // 16)
            def _(j):
                zeros_v[i, pl.ds(j * 16, 16)] = jnp.zeros((16,), F32)

        @pl.loop(0, K)
        def _(i):
            @pl.loop(0, d // 16)
            def _(j):
                ones_s[i, pl.ds(j * 16, 16)] = jnp.zeros((16,), F32)
                ones_d[i, pl.ds(j * 16, 16)] = jnp.zeros((16,), F32)
            ones_s[i, pl.ds(0, 16)] = jnp.ones((16,), F32)
            ones_d[i, pl.ds(16, 16)] = jnp.ones((16,), F32)

        base = sid * rps

        @pl.loop(0, rps // 32)
        def _(t):
            pltpu.sync_copy(zeros_v, acc_sh.at[pl.ds(base + t * 32, 32)])

        plsc.subcore_barrier()

        pltpu.sync_copy(ei_hbm.at[cid, sid, 0], idxa)

        @pl.loop(0, c_chunks // 2)
        def _(h):
            c0 = h * 2
            cb = pltpu.async_copy(ei_hbm.at[cid, sid, c0 + 1], idxb, semi)
            pltpu.sync_copy(ones_s, acc_sh.at[idxa.at[0]], add=True)
            pltpu.sync_copy(ones_d, acc_sh.at[idxa.at[1]], add=True)
            cb.wait()
            c2 = jnp.minimum(c0 + 2, c_chunks - 1)
            ca = pltpu.async_copy(ei_hbm.at[cid, sid, c2], idxa, semi)
            pltpu.sync_copy(ones_s, acc_sh.at[idxb.at[0]], add=True)
            pltpu.sync_copy(ones_d, acc_sh.at[idxb.at[1]], add=True)
            ca.wait()

        plsc.subcore_barrier()
        pltpu.sync_copy(acc_sh.at[pl.ds(base, rps)],
                        out_hbm.at[cid, pl.ds(base, rps)])

    return pl.kernel(
        body,
        out_type=jax.ShapeDtypeStruct((NC, n_pad, d), F32),
        mesh=_mesh(),
        scratch_types=[
            pltpu.VMEM((32, d), F32),
            pltpu.VMEM((K, d), F32),
            pltpu.VMEM((K, d), F32),
            pltpu.VMEM((2, K), jnp.int32),
            pltpu.VMEM((2, K), jnp.int32),
            pltpu.VMEM_SHARED((n_pad, d), F32),
            pltpu.SemaphoreType.DMA,
        ],
    )(ei)


def _sc_prop(z, ei, n_pad):
    c_chunks = ei.shape[2]
    n_groups = c_chunks // 2
    d = z.shape[1]
    rps = n_pad // NS

    def body(z_hbm, ei_hbm, out_hbm,
             zeros_v, idxa, idxb, r0, r1, acc_sh,
             semi, s0, s1):
        cid = lax.axis_index("c")
        sid = lax.axis_index("s")
        rows = [r0, r1]
        sems = [s0, s1]

        @pl.loop(0, 32)
        def _(i):
            @pl.loop(0, d // 16)
            def _(j):
                zeros_v[i, pl.ds(j * 16, 16)] = jnp.zeros((16,), F32)

        base = sid * rps

        @pl.loop(0, rps // 32)
        def _(t):
            pltpu.sync_copy(zeros_v, acc_sh.at[pl.ds(base + t * 32, 32)])

        plsc.subcore_barrier()

        pltpu.sync_copy(ei_hbm.at[cid, sid, pl.ds(0, 2)], idxa)

        def do_group(idx_v):
            descs = [pltpu.async_copy(z_hbm.at[idx_v.at[b, 0]], rows[b],
                                      sems[b]) for b in range(2)]
            for b in range(2):
                descs[b].wait()
                pltpu.sync_copy(rows[b], acc_sh.at[idx_v.at[b, 1]], add=True)

        @pl.loop(0, n_groups // 2)
        def _(h):
            g0 = h * 2
            cb = pltpu.async_copy(
                ei_hbm.at[cid, sid, pl.ds((g0 + 1) * 2, 2)], idxb, semi)
            do_group(idxa)
            cb.wait()
            g2 = jnp.minimum(g0 + 2, n_groups - 1)
            ca = pltpu.async_copy(
                ei_hbm.at[cid, sid, pl.ds(g2 * 2, 2)], idxa, semi)
            do_group(idxb)
            ca.wait()

        plsc.subcore_barrier()
        pltpu.sync_copy(acc_sh.at[pl.ds(base, rps)],
                        out_hbm.at[cid, pl.ds(base, rps)])

    return pl.kernel(
        body,
        out_type=jax.ShapeDtypeStruct((NC, n_pad, d), F32),
        mesh=_mesh(),
        scratch_types=[
            pltpu.VMEM((32, d), F32),
            pltpu.VMEM((2, 2, K), jnp.int32),
            pltpu.VMEM((2, 2, K), jnp.int32),
            pltpu.VMEM((K, d), F32),
            pltpu.VMEM((K, d), F32),
            pltpu.VMEM_SHARED((n_pad, d), F32),
            pltpu.SemaphoreType.DMA,
            pltpu.SemaphoreType.DMA,
            pltpu.SemaphoreType.DMA,
        ],
    )(z, ei)


def _bcast_lanes(v16, d):
    return jnp.concatenate([v16] * (d // 16), axis=1)


def _row_block(n):
    for br in (2000, 1000, 500, 200, 100):
        if n % br == 0 and br % 8 == 0:
            return br
    return n


def _tc_pre(x, dps, n):
    d = x.shape[1]
    br = _row_block(n)

    def body(x_ref, dp_ref, z_ref):
        deg = dp_ref[0] + dp_ref[1]
        nsrc = lax.rsqrt(jnp.maximum(deg, 1.0))
        z_ref[...] = x_ref[...] * _bcast_lanes(nsrc, d)

    return pl.pallas_call(
        body,
        grid=(n // br,),
        in_specs=[
            pl.BlockSpec((br, d), lambda i: (i, 0)),
            pl.BlockSpec((NC, br, 16), lambda i: (0, i, 0)),
        ],
        out_specs=pl.BlockSpec((br, d), lambda i: (i, 0)),
        out_shape=jax.ShapeDtypeStruct((n, d), F32),
    )(x, dps)


def _tc_layer(acc, dps, dpd, w, b, n, scale_src):
    d = w.shape[0]
    br = _row_block(n)

    def body(acc_ref, dps_ref, dpd_ref, w_ref, b_ref, o_ref):
        s = acc_ref[0] + acc_ref[1]
        ndst = lax.rsqrt(jnp.maximum(dpd_ref[0] + dpd_ref[1], 1.0))
        if scale_src:
            nsrc = lax.rsqrt(jnp.maximum(dps_ref[0] + dps_ref[1], 1.0))
            scale = ndst * nsrc
        else:
            scale = ndst
        sm = s * _bcast_lanes(scale, d)
        out = jnp.dot(sm, w_ref[...], preferred_element_type=F32,
                      precision=lax.Precision.HIGHEST)
        if scale_src:
            out = out + _bcast_lanes(nsrc, d) * b_ref[...]
        else:
            out = out + b_ref[...]
        o_ref[...] = out

    return pl.pallas_call(
        body,
        grid=(n // br,),
        in_specs=[
            pl.BlockSpec((NC, br, d), lambda i: (0, i, 0)),
            pl.BlockSpec((NC, br, 16), lambda i: (0, i, 0)),
            pl.BlockSpec((NC, br, 16), lambda i: (0, i, 0)),
            pl.BlockSpec((d, d), lambda i: (0, 0)),
            pl.BlockSpec((1, d), lambda i: (0, 0)),
        ],
        out_specs=pl.BlockSpec((br, d), lambda i: (i, 0)),
        out_shape=jax.ShapeDtypeStruct((n, d), F32),
    )(acc, dps, dpd, w, b)


def kernel(in_feat, edge_index, W1, b1, W2, b2):
    n, d = in_feat.shape
    e = edge_index.shape[1]
    src = edge_index[0]
    dst = edge_index[1]

    c_chunks = _round_up(_cdiv(_cdiv(e, NW), K), 8)
    pad = NW * c_chunks * K - e
    srcp = jnp.concatenate(
        [src, jnp.full((pad,), n, jnp.int32)]).reshape(NC, NS, c_chunks, K)
    dstp = jnp.concatenate(
        [dst, jnp.full((pad,), n, jnp.int32)]).reshape(NC, NS, c_chunks, K)
    ei = jnp.stack([srcp, dstp], axis=3)

    n_pad = _round_up(n + 1, NS * 64)
    zpad = jnp.zeros((8, d), F32)

    dp = _sc_degree(ei, n_pad, d)
    dps = dp[:, :n, 0:16]
    dpd = dp[:, :n, 16:32]
    z = _tc_pre(in_feat, dps, n)
    acc1 = _sc_prop(jnp.concatenate([z, zpad]), ei, n_pad)
    z2 = _tc_layer(acc1[:, :n, :], dps, dpd, W1, b1.reshape(1, d), n,
                   scale_src=True)
    acc2 = _sc_prop(jnp.concatenate([z2, zpad]), ei, n_pad)
    out = _tc_layer(acc2[:, :n, :], dps, dpd, W2, b2.reshape(1, d), n,
                    scale_src=False)
    return out


# speedup vs baseline: 1.9902x; 1.9902x over previous
import jax
import jax.numpy as jnp
from jax import lax
from jax.experimental import pallas as pl
from jax.experimental.pallas import tpu as pltpu
from jax.experimental.pallas import tpu_sc as plsc

NC = 2
NS = 16
NW = NC * NS
K = 128
F32 = jnp.float32


def _mesh():
    return plsc.VectorSubcoreMesh(core_axis_name="c", subcore_axis_name="s",
                                  num_cores=NC, num_subcores=NS)


def _cdiv(a, b):
    return -(-a // b)


def _round_up(a, b):
    return _cdiv(a, b) * b


def _sc_degree(ei, n_pad, d):
    c_chunks = ei.shape[2]
    rps = n_pad // NS

    def body(ei_hbm, out_hbm, zeros_v, ones_s, ones_d, idxa, idxb, acc_sh,
             semi):
        cid = lax.axis_index("c")
        sid = lax.axis_index("s")

        @pl.loop(0, 32)
        def _(i):
            @pl.loop(0, d // 16)
            def _(j):
                zeros_v[i, pl.ds(j * 16, 16)] = jnp.zeros((16,), F32)

        @pl.loop(0, K)
        def _(i):
            @pl.loop(0, d // 16)
            def _(j):
                ones_s[i, pl.ds(j * 16, 16)] = jnp.zeros((16,), F32)
                ones_d[i, pl.ds(j * 16, 16)] = jnp.zeros((16,), F32)
            ones_s[i, pl.ds(0, 16)] = jnp.ones((16,), F32)
            ones_d[i, pl.ds(16, 16)] = jnp.ones((16,), F32)

        base = sid * rps

        @pl.loop(0, rps // 32)
        def _(t):
            pltpu.sync_copy(zeros_v, acc_sh.at[pl.ds(base + t * 32, 32)])

        plsc.subcore_barrier()

        pltpu.sync_copy(ei_hbm.at[cid, sid, 0], idxa)

        @pl.loop(0, c_chunks // 2)
        def _(h):
            c0 = h * 2
            cb = pltpu.async_copy(ei_hbm.at[cid, sid, c0 + 1], idxb, semi)
            pltpu.sync_copy(ones_s, acc_sh.at[idxa.at[0]], add=True)
            pltpu.sync_copy(ones_d, acc_sh.at[idxa.at[1]], add=True)
            cb.wait()
            c2 = jnp.minimum(c0 + 2, c_chunks - 1)
            ca = pltpu.async_copy(ei_hbm.at[cid, sid, c2], idxa, semi)
            pltpu.sync_copy(ones_s, acc_sh.at[idxb.at[0]], add=True)
            pltpu.sync_copy(ones_d, acc_sh.at[idxb.at[1]], add=True)
            ca.wait()

        plsc.subcore_barrier()
        pltpu.sync_copy(acc_sh.at[pl.ds(base, rps)],
                        out_hbm.at[cid, pl.ds(base, rps)])

    return pl.kernel(
        body,
        out_type=jax.ShapeDtypeStruct((NC, n_pad, d), F32),
        mesh=_mesh(),
        scratch_types=[
            pltpu.VMEM((32, d), F32),
            pltpu.VMEM((K, d), F32),
            pltpu.VMEM((K, d), F32),
            pltpu.VMEM((2, K), jnp.int32),
            pltpu.VMEM((2, K), jnp.int32),
            pltpu.VMEM_SHARED((n_pad, d), F32),
            pltpu.SemaphoreType.DMA,
        ],
    )(ei)


def _sc_prop(z, ei, n_pad):
    c_chunks = ei.shape[2]
    n_groups = c_chunks // 2
    d = z.shape[1]
    rps = n_pad // NS

    def body(z_hbm, ei_hbm, out_hbm,
             zeros_v, idxa, idxb, r0, r1, acc_sh,
             semi, s0, s1):
        cid = lax.axis_index("c")
        sid = lax.axis_index("s")
        rows = [r0, r1]
        sems = [s0, s1]

        @pl.loop(0, 32)
        def _(i):
            @pl.loop(0, d // 16)
            def _(j):
                zeros_v[i, pl.ds(j * 16, 16)] = jnp.zeros((16,), F32)

        base = sid * rps

        @pl.loop(0, rps // 32)
        def _(t):
            pltpu.sync_copy(zeros_v, acc_sh.at[pl.ds(base + t * 32, 32)])

        plsc.subcore_barrier()

        pltpu.sync_copy(ei_hbm.at[cid, sid, pl.ds(0, 2)], idxa)

        def do_group(idx_v):
            descs = [pltpu.async_copy(z_hbm.at[idx_v.at[b, 0]], rows[b],
                                      sems[b]) for b in range(2)]
            for b in range(2):
                descs[b].wait()
                pltpu.sync_copy(rows[b], acc_sh.at[idx_v.at[b, 1]], add=True)

        @pl.loop(0, n_groups // 2)
        def _(h):
            g0 = h * 2
            cb = pltpu.async_copy(
                ei_hbm.at[cid, sid, pl.ds((g0 + 1) * 2, 2)], idxb, semi)
            do_group(idxa)
            cb.wait()
            g2 = jnp.minimum(g0 + 2, n_groups - 1)
            ca = pltpu.async_copy(
                ei_hbm.at[cid, sid, pl.ds(g2 * 2, 2)], idxa, semi)
            do_group(idxb)
            ca.wait()

        plsc.subcore_barrier()
        pltpu.sync_copy(acc_sh.at[pl.ds(base, rps)],
                        out_hbm.at[cid, pl.ds(base, rps)])

    return pl.kernel(
        body,
        out_type=jax.ShapeDtypeStruct((NC, n_pad, d), F32),
        mesh=_mesh(),
        scratch_types=[
            pltpu.VMEM((32, d), F32),
            pltpu.VMEM((2, 2, K), jnp.int32),
            pltpu.VMEM((2, 2, K), jnp.int32),
            pltpu.VMEM((K, d), F32),
            pltpu.VMEM((K, d), F32),
            pltpu.VMEM_SHARED((n_pad, d), F32),
            pltpu.SemaphoreType.DMA,
            pltpu.SemaphoreType.DMA,
            pltpu.SemaphoreType.DMA,
        ],
    )(z, ei)


def _bcast_lanes(v16, d):
    return jnp.concatenate([v16] * (d // 16), axis=1)


def _row_block(n):
    for br in (2000, 1000, 500, 200, 100):
        if n % br == 0 and br % 8 == 0:
            return br
    return n


def _tc_pre(x, dps, n):
    d = x.shape[1]
    br = _row_block(n)

    def body(x_ref, dp_ref, z_ref):
        deg = dp_ref[0] + dp_ref[1]
        nsrc = lax.rsqrt(jnp.maximum(deg, 1.0))
        z_ref[...] = x_ref[...] * _bcast_lanes(nsrc, d)

    return pl.pallas_call(
        body,
        grid=(n // br,),
        in_specs=[
            pl.BlockSpec((br, d), lambda i: (i, 0)),
            pl.BlockSpec((NC, br, 16), lambda i: (0, i, 0)),
        ],
        out_specs=pl.BlockSpec((br, d), lambda i: (i, 0)),
        out_shape=jax.ShapeDtypeStruct((n, d), F32),
    )(x, dps)


def _tc_layer(acc, dps, dpd, w, b, n, scale_src):
    d = w.shape[0]
    br = _row_block(n)

    def body(acc_ref, dps_ref, dpd_ref, w_ref, b_ref, o_ref):
        s = acc_ref[0] + acc_ref[1]
        ndst = lax.rsqrt(jnp.maximum(dpd_ref[0] + dpd_ref[1], 1.0))
        if scale_src:
            nsrc = lax.rsqrt(jnp.maximum(dps_ref[0] + dps_ref[1], 1.0))
            scale = ndst * nsrc
        else:
            scale = ndst
        sm = s * _bcast_lanes(scale, d)
        out = jnp.dot(sm, w_ref[...], preferred_element_type=F32,
                      precision=lax.Precision.HIGHEST)
        if scale_src:
            out = out + _bcast_lanes(nsrc, d) * b_ref[...]
        else:
            out = out + b_ref[...]
        o_ref[...] = out

    return pl.pallas_call(
        body,
        grid=(n // br,),
        in_specs=[
            pl.BlockSpec((NC, br, d), lambda i: (0, i, 0)),
            pl.BlockSpec((NC, br, 16), lambda i: (0, i, 0)),
            pl.BlockSpec((NC, br, 16), lambda i: (0, i, 0)),
            pl.BlockSpec((d, d), lambda i: (0, 0)),
            pl.BlockSpec((1, d), lambda i: (0, 0)),
        ],
        out_specs=pl.BlockSpec((br, d), lambda i: (i, 0)),
        out_shape=jax.ShapeDtypeStruct((n, d), F32),
    )(acc, dps, dpd, w, b)


def kernel(in_feat, edge_index, W1, b1, W2, b2):
    n, d = in_feat.shape
    e = edge_index.shape[1]
    src = edge_index[0]
    dst = edge_index[1]

    c_chunks = _round_up(_cdiv(_cdiv(e, NW), K), 8)
    pad = NW * c_chunks * K - e
    n_pad = _round_up(n + 1, NS * 64)
    spare = n_pad - n
    pad_src = n + (jnp.arange(pad, dtype=jnp.int32) % 8)
    pad_dst = n + (jnp.arange(pad, dtype=jnp.int32) % spare)
    srcp = jnp.concatenate([src, pad_src]).reshape(NC, NS, c_chunks, K)
    dstp = jnp.concatenate([dst, pad_dst]).reshape(NC, NS, c_chunks, K)
    ei = jnp.stack([srcp, dstp], axis=3)

    zpad = jnp.zeros((8, d), F32)

    dp = _sc_degree(ei, n_pad, d)
    dps = dp[:, :n, 0:16]
    dpd = dp[:, :n, 16:32]
    z = _tc_pre(in_feat, dps, n)
    acc1 = _sc_prop(jnp.concatenate([z, zpad]), ei, n_pad)
    z2 = _tc_layer(acc1[:, :n, :], dps, dpd, W1, b1.reshape(1, d), n,
                   scale_src=True)
    acc2 = _sc_prop(jnp.concatenate([z2, zpad]), ei, n_pad)
    out = _tc_layer(acc2[:, :n, :], dps, dpd, W2, b2.reshape(1, d), n,
                    scale_src=False)
    return out
